# Initial kernel scaffold; baseline (speedup 1.0000x reference)
import jax
import jax.numpy as jnp
from jax.experimental import pallas as pl


def kernel(hidden_states, weight, e_score_correction_bias):
    raise NotImplementedError("write your pallas kernel here")


# speedup vs baseline: 1.6669x; 1.6669x over previous
import jax
import jax.numpy as jnp
from jax.experimental import pallas as pl
from jax.experimental.pallas import tpu as pltpu

HIDDEN = 4096
N_EXPERTS = 128
TOP_K = 8
N_TOK = 32768
BT = 256


def _router_body(hs_ref, wt_ref, bias_ref, idx_ref, w_ref):
    logits = jnp.dot(hs_ref[...], wt_ref[...],
                     preferred_element_type=jnp.float32)
    scores = jax.nn.sigmoid(logits)
    choice = scores + bias_ref[...]

    col = jax.lax.broadcasted_iota(jnp.int32, (BT, N_EXPERTS), 1)
    x = choice
    idx_list = []
    val_list = []
    for _ in range(TOP_K):
        m = jnp.max(x, axis=1, keepdims=True)
        is_max = x == m
        idx = jnp.min(jnp.where(is_max, col, N_EXPERTS), axis=1, keepdims=True)
        onehot = col == idx
        val = jnp.sum(jnp.where(onehot, scores, 0.0), axis=1, keepdims=True)
        x = jnp.where(onehot, -jnp.inf, x)
        idx_list.append(idx)
        val_list.append(val)
    inds = jnp.concatenate(idx_list, axis=1)
    vals = jnp.concatenate(val_list, axis=1)
    denom = jnp.sum(vals, axis=1, keepdims=True) + 1e-20
    idx_ref[...] = inds
    w_ref[...] = vals / denom


def kernel(hidden_states, weight, e_score_correction_bias):
    hs = hidden_states.reshape(-1, HIDDEN).astype(jnp.float32)
    wt = weight.astype(jnp.float32).T
    bias = e_score_correction_bias.reshape(1, N_EXPERTS).astype(jnp.float32)

    grid = (N_TOK // BT,)
    out_shape = (
        jax.ShapeDtypeStruct((N_TOK, TOP_K), jnp.int32),
        jax.ShapeDtypeStruct((N_TOK, TOP_K), jnp.float32),
    )
    topk_indices, topk_weights = pl.pallas_call(
        _router_body,
        grid=grid,
        in_specs=[
            pl.BlockSpec((BT, HIDDEN), lambda i: (i, 0)),
            pl.BlockSpec((HIDDEN, N_EXPERTS), lambda i: (0, 0)),
            pl.BlockSpec((1, N_EXPERTS), lambda i: (0, 0)),
        ],
        out_specs=(
            pl.BlockSpec((BT, TOP_K), lambda i: (i, 0)),
            pl.BlockSpec((BT, TOP_K), lambda i: (i, 0)),
        ),
        out_shape=out_shape,
    )(hs, wt, bias)
    return topk_indices, topk_weights
